# Initial kernel scaffold; baseline (speedup 1.0000x reference)
import jax
import jax.numpy as jnp
from jax.experimental import pallas as pl


def kernel(x1, x2, ei_intra1, ei_intra2, ei_inter, W_intra1, a_src_intra1, a_dst_intra1, b_intra1, W_intra2, a_src_intra2, a_dst_intra2, b_intra2, Ws_inter, Wd_inter, a_src_inter, a_dst_inter, b_inter, W_lin, b_lin):
    raise NotImplementedError("write your pallas kernel here")


# speedup vs baseline: 27.6348x; 27.6348x over previous
import functools

import jax
import jax.numpy as jnp
from jax import lax
from jax.experimental import pallas as pl
from jax.experimental.pallas import tpu as pltpu
from jax.experimental.pallas import tpu_sc as plsc

C = 128
CP = 144
K = 128
NC = 2
NS = 16
NW = NC * NS


def _proj_body(x_ref, w_ref, a_ref, hp_ref, sd_ref):
    x = x_ref[...]
    br = x.shape[0]
    h = lax.dot_general(x, w_ref[...], (((1,), (1,)), ((), ())),
                        preferred_element_type=jnp.float32)
    sd = lax.dot_general(h, a_ref[...], (((1,), (1,)), ((), ())),
                         preferred_element_type=jnp.float32)
    lane = lax.broadcasted_iota(jnp.int32, (br, CP - C), 1)
    pad = jnp.where(lane == 0, 1.0, jnp.where(lane == 1, sd[:, 0:1], 0.0))
    hp_ref[...] = jnp.concatenate([h, pad], axis=1)
    sd_ref[...] = sd


def _project(x, W, a_s, a_d, br=2000):
    n = x.shape[0]
    a = jnp.stack([a_s, a_d])
    return pl.pallas_call(
        _proj_body,
        grid=(n // br,),
        in_specs=[pl.BlockSpec((br, C), lambda i: (i, 0)),
                  pl.BlockSpec((C, C), lambda i: (0, 0)),
                  pl.BlockSpec((2, C), lambda i: (0, 0))],
        out_specs=[pl.BlockSpec((br, CP), lambda i: (i, 0)),
                   pl.BlockSpec((br, 2), lambda i: (i, 0))],
        out_shape=[jax.ShapeDtypeStruct((n, CP), jnp.float32),
                   jax.ShapeDtypeStruct((n, 2), jnp.float32)],
    )(x, W, a)


def _proj_sd_body(x_ref, w_ref, a_ref, sd_ref):
    h = lax.dot_general(x_ref[...], w_ref[...], (((1,), (1,)), ((), ())),
                        preferred_element_type=jnp.float32)
    sd_ref[...] = lax.dot_general(h, a_ref[...], (((1,), (1,)), ((), ())),
                                  preferred_element_type=jnp.float32)


def _project_sd(x, W, a_s, a_d, br=2000):
    n = x.shape[0]
    a = jnp.stack([a_s, a_d])
    return pl.pallas_call(
        _proj_sd_body,
        grid=(n // br,),
        in_specs=[pl.BlockSpec((br, C), lambda i: (i, 0)),
                  pl.BlockSpec((C, C), lambda i: (0, 0)),
                  pl.BlockSpec((2, C), lambda i: (0, 0))],
        out_specs=pl.BlockSpec((br, 2), lambda i: (i, 0)),
        out_shape=jax.ShapeDtypeStruct((n, 2), jnp.float32),
    )(x, W, a)


def _comb1_body(acc_ref, b_ref, out_ref):
    a = acc_ref[0] + acc_ref[1]
    num = a[:, :C]
    den = a[:, C:C + 1]
    out_ref[...] = jnp.maximum(num / (den + 1e-16) + b_ref[...], 0.0)


def _combine1(acc, b, br=2000):
    n = acc.shape[1]
    return pl.pallas_call(
        _comb1_body,
        grid=(n // br,),
        in_specs=[pl.BlockSpec((2, br, CP), lambda i: (0, i, 0)),
                  pl.BlockSpec((1, C), lambda i: (0, 0))],
        out_specs=pl.BlockSpec((br, C), lambda i: (i, 0)),
        out_shape=jax.ShapeDtypeStruct((n, C), jnp.float32),
    )(acc, b.reshape(1, C))


def _comb2_body(acca_ref, accb_ref, ba_ref, bb_ref, out_ref):
    a = acca_ref[0] + acca_ref[1]
    oa = a[:, :C] / (a[:, C:C + 1] + 1e-16) + ba_ref[...]
    b = accb_ref[0] + accb_ref[1]
    ob = b[:, :C] / (b[:, C:C + 1] + 1e-16) + bb_ref[...]
    out_ref[...] = jnp.maximum(oa + ob, 0.0)


def _combine2(acca, accb, ba, bb, br=2000):
    n = acca.shape[1]
    return pl.pallas_call(
        _comb2_body,
        grid=(n // br,),
        in_specs=[pl.BlockSpec((2, br, CP), lambda i: (0, i, 0)),
                  pl.BlockSpec((2, br, CP), lambda i: (0, i, 0)),
                  pl.BlockSpec((1, C), lambda i: (0, 0)),
                  pl.BlockSpec((1, C), lambda i: (0, 0))],
        out_specs=pl.BlockSpec((br, C), lambda i: (i, 0)),
        out_shape=jax.ShapeDtypeStruct((n, C), jnp.float32),
    )(acca, accb, ba.reshape(1, C), bb.reshape(1, C))


def _lin_body(x_ref, w_ref, b_ref, out_ref):
    out_ref[...] = lax.dot_general(x_ref[...], w_ref[...],
                                   (((1,), (1,)), ((), ())),
                                   preferred_element_type=jnp.float32) + b_ref[...]


def _linear(x, W, b, br=2000):
    n = x.shape[0]
    return pl.pallas_call(
        _lin_body,
        grid=(n // br,),
        in_specs=[pl.BlockSpec((br, C), lambda i: (i, 0)),
                  pl.BlockSpec((C, C), lambda i: (0, 0)),
                  pl.BlockSpec((1, C), lambda i: (0, 0))],
        out_specs=pl.BlockSpec((br, C), lambda i: (i, 0)),
        out_shape=jax.ShapeDtypeStruct((n, C), jnp.float32),
    )(x, W, b.reshape(1, C))


@functools.cache
def _make_edge_kernel(E, ns, nd):
    assert E % K == 0 and nd % NS == 0
    nchunks = E // K
    rpt = nd // NS

    pieces = []
    off = 0
    while off < rpt:
        cnt = min(K, rpt - off)
        pieces.append((off, cnt))
        off += cnt

    mesh = plsc.VectorSubcoreMesh(core_axis_name="c", subcore_axis_name="s")

    @functools.partial(
        pl.kernel, mesh=mesh,
        compiler_params=pltpu.CompilerParams(use_tc_tiling_on_sc=False,
                                             needs_layout_passes=False),
        out_type=jax.ShapeDtypeStruct((NC, nd, CP), jnp.float32),
        scratch_types=[
            pltpu.VMEM((nd,), jnp.float32),
            pltpu.VMEM((K,), jnp.int32),
            pltpu.VMEM((K,), jnp.int32),
            pltpu.VMEM((K,), jnp.float32),
            pltpu.VMEM((K, CP), jnp.float32),
            pltpu.VMEM_SHARED((nd, CP), jnp.float32),
            pltpu.SemaphoreType.DMA,
        ],
    )
    def edge_kernel(hs_hbm, d_hbm, src_hbm, dst_hbm, out_hbm,
                    d_v, src_v, dst_v, w_v, rows_v, acc_sh, sem):
        cid = lax.axis_index("c")
        sid = lax.axis_index("s")
        wid = sid * NC + cid

        pltpu.sync_copy(d_hbm, d_v)

        zero16 = jnp.zeros((16,), jnp.float32)

        def zbody(e, carry):
            for j in range(CP // 16):
                rows_v[e, pl.ds(j * 16, 16)] = zero16
            return carry

        lax.fori_loop(0, K, zbody, 0)
        base_row = sid * rpt
        for off2, cnt in pieces:
            pltpu.sync_copy(rows_v.at[pl.ds(0, cnt)],
                            acc_sh.at[pl.ds(base_row + off2, cnt)])
        plsc.subcore_barrier()

        iota16 = lax.iota(jnp.int32, 16)
        col_s = jnp.full((16,), C + 1, jnp.int32)

        def body(i, carry):
            base = (wid + i * NW) * K
            pltpu.sync_copy(src_hbm.at[pl.ds(base, K)], src_v)
            pltpu.sync_copy(dst_hbm.at[pl.ds(base, K)], dst_v)
            pltpu.async_copy(hs_hbm.at[src_v], rows_v, sem).wait()
            for i16 in range(K // 16):
                di = dst_v[pl.ds(i16 * 16, 16)]
                sval = plsc.load_gather(rows_v, [iota16 + (i16 * 16), col_s])
                dval = plsc.load_gather(d_v, [di])
                al = sval + dval
                al = jnp.where(al >= 0.0, al, al * 0.2)
                w_v[pl.ds(i16 * 16, 16)] = jnp.exp(al)

            def scale(g, c2):
                w16 = w_v[pl.ds(g * 16, 16)]
                for e16 in range(16):
                    e = g * 16 + e16
                    w = w16[e16]
                    for j in range(CP // 16):
                        rows_v[e, pl.ds(j * 16, 16)] = rows_v[e, pl.ds(j * 16, 16)] * w
                return c2

            lax.fori_loop(0, K // 16, scale, 0)
            pltpu.sync_copy(rows_v, acc_sh.at[dst_v], add=True)
            return carry

        nchunks_w = (nchunks - wid + NW - 1) // NW
        lax.fori_loop(0, nchunks_w, body, 0)
        plsc.subcore_barrier()

        for off2, cnt in pieces:
            pltpu.sync_copy(acc_sh.at[pl.ds(base_row + off2, cnt)],
                            rows_v.at[pl.ds(0, cnt)])
            pltpu.sync_copy(rows_v.at[pl.ds(0, cnt)],
                            out_hbm.at[cid, pl.ds(base_row + off2, cnt)])

    return edge_kernel


def kernel(x1, x2, ei_intra1, ei_intra2, ei_inter,
           W_intra1, a_src_intra1, a_dst_intra1, b_intra1,
           W_intra2, a_src_intra2, a_dst_intra2, b_intra2,
           Ws_inter, Wd_inter, a_src_inter, a_dst_inter, b_inter,
           W_lin, b_lin):
    n1, n2 = x1.shape[0], x2.shape[0]
    e = ei_intra1.shape[1]
    edge = _make_edge_kernel(e, n1, n2)

    h1, h2 = x1, x2
    for l in range(W_intra1.shape[0]):
        hp1, sd1 = _project(h1, W_intra1[l], a_src_intra1[l], a_dst_intra1[l])
        hp2, sd2 = _project(h2, W_intra2[l], a_src_intra2[l], a_dst_intra2[l])
        hp3, _ = _project(h1, Ws_inter[l], a_src_inter[l], a_dst_inter[l])
        sd3d = _project_sd(h2, Wd_inter[l], a_src_inter[l], a_dst_inter[l])

        acc1 = edge(hp1, sd1[:, 1], ei_intra1[0], ei_intra1[1])
        acc2 = edge(hp2, sd2[:, 1], ei_intra2[0], ei_intra2[1])
        acc3 = edge(hp3, sd3d[:, 1], ei_inter[0], ei_inter[1])

        h1 = _combine1(acc1, b_intra1[l])
        h2 = _combine2(acc2, acc3, b_intra2[l], b_inter[l])

    return _linear(h1, W_lin, b_lin), _linear(h2, W_lin, b_lin)


# speedup vs baseline: 47.8916x; 1.7330x over previous
import functools

import jax
import jax.numpy as jnp
from jax import lax
from jax.experimental import pallas as pl
from jax.experimental.pallas import tpu as pltpu
from jax.experimental.pallas import tpu_sc as plsc

C = 128
CP = 144
K = 64
NC = 2
NS = 16
NW = NC * NS


def _proj_body(x_ref, w_ref, a_ref, hp_ref, sd_ref):
    x = x_ref[...]
    br = x.shape[0]
    h = lax.dot_general(x, w_ref[...], (((1,), (1,)), ((), ())),
                        preferred_element_type=jnp.float32)
    sd = lax.dot_general(h, a_ref[...], (((1,), (1,)), ((), ())),
                         preferred_element_type=jnp.float32)
    lane = lax.broadcasted_iota(jnp.int32, (br, CP - C), 1)
    pad = jnp.where(lane == 0, 1.0, jnp.where(lane == 1, sd[:, 0:1], 0.0))
    hp_ref[...] = jnp.concatenate([h, pad], axis=1)
    sd_ref[...] = jnp.concatenate(
        [sd, jnp.zeros((br, 14), jnp.float32)], axis=1)


def _project(x, W, a_s, a_d, br=2000):
    n = x.shape[0]
    a = jnp.stack([a_s, a_d])
    return pl.pallas_call(
        _proj_body,
        grid=(n // br,),
        in_specs=[pl.BlockSpec((br, C), lambda i: (i, 0)),
                  pl.BlockSpec((C, C), lambda i: (0, 0)),
                  pl.BlockSpec((2, C), lambda i: (0, 0))],
        out_specs=[pl.BlockSpec((br, CP), lambda i: (i, 0)),
                   pl.BlockSpec((br, 16), lambda i: (i, 0))],
        out_shape=[jax.ShapeDtypeStruct((n, CP), jnp.float32),
                   jax.ShapeDtypeStruct((n, 16), jnp.float32)],
    )(x, W, a)


def _proj_sd_body(x_ref, w_ref, a_ref, sd_ref):
    h = lax.dot_general(x_ref[...], w_ref[...], (((1,), (1,)), ((), ())),
                        preferred_element_type=jnp.float32)
    sd = lax.dot_general(h, a_ref[...], (((1,), (1,)), ((), ())),
                         preferred_element_type=jnp.float32)
    br = sd.shape[0]
    sd_ref[...] = jnp.concatenate(
        [sd, jnp.zeros((br, 14), jnp.float32)], axis=1)


def _project_sd(x, W, a_s, a_d, br=2000):
    n = x.shape[0]
    a = jnp.stack([a_s, a_d])
    return pl.pallas_call(
        _proj_sd_body,
        grid=(n // br,),
        in_specs=[pl.BlockSpec((br, C), lambda i: (i, 0)),
                  pl.BlockSpec((C, C), lambda i: (0, 0)),
                  pl.BlockSpec((2, C), lambda i: (0, 0))],
        out_specs=pl.BlockSpec((br, 16), lambda i: (i, 0)),
        out_shape=jax.ShapeDtypeStruct((n, 16), jnp.float32),
    )(x, W, a)


def _comb1_body(acc_ref, b_ref, out_ref):
    a = acc_ref[0] + acc_ref[1]
    num = a[:, :C]
    den = a[:, C:C + 1]
    out_ref[...] = jnp.maximum(num / (den + 1e-16) + b_ref[...], 0.0)


def _combine1(acc, b, br=2000):
    n = acc.shape[1]
    return pl.pallas_call(
        _comb1_body,
        grid=(n // br,),
        in_specs=[pl.BlockSpec((2, br, CP), lambda i: (0, i, 0)),
                  pl.BlockSpec((1, C), lambda i: (0, 0))],
        out_specs=pl.BlockSpec((br, C), lambda i: (i, 0)),
        out_shape=jax.ShapeDtypeStruct((n, C), jnp.float32),
    )(acc, b.reshape(1, C))


def _comb2_body(acca_ref, accb_ref, ba_ref, bb_ref, out_ref):
    a = acca_ref[0] + acca_ref[1]
    oa = a[:, :C] / (a[:, C:C + 1] + 1e-16) + ba_ref[...]
    b = accb_ref[0] + accb_ref[1]
    ob = b[:, :C] / (b[:, C:C + 1] + 1e-16) + bb_ref[...]
    out_ref[...] = jnp.maximum(oa + ob, 0.0)


def _combine2(acca, accb, ba, bb, br=2000):
    n = acca.shape[1]
    return pl.pallas_call(
        _comb2_body,
        grid=(n // br,),
        in_specs=[pl.BlockSpec((2, br, CP), lambda i: (0, i, 0)),
                  pl.BlockSpec((2, br, CP), lambda i: (0, i, 0)),
                  pl.BlockSpec((1, C), lambda i: (0, 0)),
                  pl.BlockSpec((1, C), lambda i: (0, 0))],
        out_specs=pl.BlockSpec((br, C), lambda i: (i, 0)),
        out_shape=jax.ShapeDtypeStruct((n, C), jnp.float32),
    )(acca, accb, ba.reshape(1, C), bb.reshape(1, C))


def _lin_body(x_ref, w_ref, b_ref, out_ref):
    out_ref[...] = lax.dot_general(x_ref[...], w_ref[...],
                                   (((1,), (1,)), ((), ())),
                                   preferred_element_type=jnp.float32) + b_ref[...]


def _linear(x, W, b, br=2000):
    n = x.shape[0]
    return pl.pallas_call(
        _lin_body,
        grid=(n // br,),
        in_specs=[pl.BlockSpec((br, C), lambda i: (i, 0)),
                  pl.BlockSpec((C, C), lambda i: (0, 0)),
                  pl.BlockSpec((1, C), lambda i: (0, 0))],
        out_specs=pl.BlockSpec((br, C), lambda i: (i, 0)),
        out_shape=jax.ShapeDtypeStruct((n, C), jnp.float32),
    )(x, W, b.reshape(1, C))


@functools.cache
def _make_edge_kernel(E, ns, nd):
    assert E % K == 0 and nd % NS == 0
    nchunks = E // K
    base_rows = nchunks // NW
    extra = nchunks - base_rows * NW
    rmax = base_rows + 1
    first_extra = NW - extra
    rpt = nd // NS

    pieces = []
    off = 0
    while off < rpt:
        cnt = min(K, rpt - off)
        pieces.append((off, cnt))
        off += cnt

    mesh = plsc.VectorSubcoreMesh(core_axis_name="c", subcore_axis_name="s")

    @functools.partial(
        pl.kernel, mesh=mesh,
        compiler_params=pltpu.CompilerParams(use_tc_tiling_on_sc=False,
                                             needs_layout_passes=False),
        out_type=jax.ShapeDtypeStruct((NC, nd, CP), jnp.float32),
        scratch_types=[
            pltpu.VMEM((rmax, K), jnp.int32),
            pltpu.VMEM((rmax, K), jnp.int32),
            pltpu.VMEM((K, CP), jnp.float32),
            pltpu.VMEM((K, CP), jnp.float32),
            pltpu.VMEM((K, 16), jnp.float32),
            pltpu.VMEM((K, 16), jnp.float32),
            pltpu.VMEM((K,), jnp.int32),
            pltpu.VMEM((K,), jnp.int32),
            pltpu.VMEM((K,), jnp.int32),
            pltpu.VMEM((K,), jnp.int32),
            pltpu.VMEM_SHARED((nd, CP), jnp.float32),
            pltpu.SemaphoreType.DMA,
            pltpu.SemaphoreType.DMA,
            pltpu.SemaphoreType.DMA,
            pltpu.SemaphoreType.DMA,
        ],
    )
    def edge_kernel(hs_hbm, sd_hbm, src_hbm, dst_hbm, out_hbm,
                    src_v, dst_v, rows0, rows1, rd0, rd1,
                    sc0, sc1, dc0, dc1, acc_sh,
                    gsem0, gsem1, ssem0, ssem1):
        rows = (rows0, rows1)
        rd = (rd0, rd1)
        src_c = (sc0, sc1)
        dst_c = (dc0, dc1)
        gsem = (gsem0, gsem1)
        ssem = (ssem0, ssem1)
        cid = lax.axis_index("c")
        sid = lax.axis_index("s")
        wid = sid * NC + cid

        r0 = wid * base_rows + jnp.maximum(wid - first_extra, 0)
        nrows = base_rows + (wid >= first_extra).astype(jnp.int32)
        pltpu.sync_copy(src_hbm.at[pl.ds(r0, rmax)], src_v)
        pltpu.sync_copy(dst_hbm.at[pl.ds(r0, rmax)], dst_v)

        zero16 = jnp.zeros((16,), jnp.float32)

        def zbody(e, carry):
            for j in range(CP // 16):
                rows0[e, pl.ds(j * 16, 16)] = zero16
            return carry

        lax.fori_loop(0, K, zbody, 0)
        base_row = sid * rpt
        for off2, cnt in pieces:
            pltpu.sync_copy(rows0.at[pl.ds(0, cnt)],
                            acc_sh.at[pl.ds(base_row + off2, cnt)])
        plsc.subcore_barrier()

        iota16 = lax.iota(jnp.int32, 16)
        col_s = jnp.full((16,), C + 1, jnp.int32)
        col_d = jnp.ones((16,), jnp.int32)

        def issue_gather(c, b):
            for g in range(K // 16):
                src_c[b][pl.ds(g * 16, 16)] = src_v[c, pl.ds(g * 16, 16)]
                dst_c[b][pl.ds(g * 16, 16)] = dst_v[c, pl.ds(g * 16, 16)]
            pltpu.async_copy(hs_hbm.at[src_c[b], :], rows[b], gsem[b])
            pltpu.async_copy(sd_hbm.at[dst_c[b], :], rd[b], gsem[b])

        def wait_gather(b):
            pltpu.make_async_copy(hs_hbm.at[src_c[b], :], rows[b], gsem[b]).wait()
            pltpu.make_async_copy(sd_hbm.at[dst_c[b], :], rd[b], gsem[b]).wait()

        def issue_scatter(b):
            pltpu.async_copy(rows[b], acc_sh.at[dst_c[b], :], ssem[b], add=True)

        def wait_scatter(b):
            pltpu.make_async_copy(rows[b], acc_sh.at[dst_c[b], :], ssem[b]).wait()

        def process(c, b):
            wait_gather(b)

            def group(g, carry):
                sval = plsc.load_gather(rows[b], [iota16 + g * 16, col_s])
                dval = plsc.load_gather(rd[b], [iota16 + g * 16, col_d])
                al = sval + dval
                al = jnp.where(al >= 0.0, al, al * 0.2)
                w16 = jnp.exp(al)
                for e16 in range(16):
                    e = g * 16 + e16
                    w = w16[e16]
                    for j in range(CP // 16):
                        rows[b][e, pl.ds(j * 16, 16)] = (
                            rows[b][e, pl.ds(j * 16, 16)] * w)
                return carry

            lax.fori_loop(0, K // 16, group, 0)
            issue_scatter(b)

        issue_gather(0, 0)

        def loop_body(cc, carry):
            for sub in range(2):
                c = cc * 2 + sub

                @pl.when(c < nrows)
                def _():
                    @pl.when(c + 1 < nrows)
                    def _():
                        @pl.when(c >= 1)
                        def _():
                            wait_scatter(1 - sub)
                        issue_gather(c + 1, 1 - sub)
                    process(c, sub)
            return carry

        lax.fori_loop(0, (nrows + 1) // 2, loop_body, 0)
        wait_scatter(0)
        wait_scatter(1)
        plsc.subcore_barrier()

        for off2, cnt in pieces:
            pltpu.sync_copy(acc_sh.at[pl.ds(base_row + off2, cnt)],
                            rows0.at[pl.ds(0, cnt)])
            pltpu.sync_copy(rows0.at[pl.ds(0, cnt)],
                            out_hbm.at[cid, pl.ds(base_row + off2, cnt)])

    return edge_kernel


def kernel(x1, x2, ei_intra1, ei_intra2, ei_inter,
           W_intra1, a_src_intra1, a_dst_intra1, b_intra1,
           W_intra2, a_src_intra2, a_dst_intra2, b_intra2,
           Ws_inter, Wd_inter, a_src_inter, a_dst_inter, b_inter,
           W_lin, b_lin):
    n1, n2 = x1.shape[0], x2.shape[0]
    e = ei_intra1.shape[1]
    edge = _make_edge_kernel(e, n1, n2)

    def split_ei(ei):
        return ei[0].reshape(e // K, K), ei[1].reshape(e // K, K)

    s1, d1 = split_ei(ei_intra1)
    s2, d2 = split_ei(ei_intra2)
    s3, d3 = split_ei(ei_inter)

    h1, h2 = x1, x2
    for l in range(W_intra1.shape[0]):
        hp1, sd1 = _project(h1, W_intra1[l], a_src_intra1[l], a_dst_intra1[l])
        hp2, sd2 = _project(h2, W_intra2[l], a_src_intra2[l], a_dst_intra2[l])
        hp3, _ = _project(h1, Ws_inter[l], a_src_inter[l], a_dst_inter[l])
        sd3d = _project_sd(h2, Wd_inter[l], a_src_inter[l], a_dst_inter[l])

        acc1 = edge(hp1, sd1, s1, d1)
        acc2 = edge(hp2, sd2, s2, d2)
        acc3 = edge(hp3, sd3d, s3, d3)

        h1 = _combine1(acc1, b_intra1[l])
        h2 = _combine2(acc2, acc3, b_intra2[l], b_inter[l])

    return _linear(h1, W_lin, b_lin), _linear(h2, W_lin, b_lin)
